# baseline (speedup 1.0000x reference)
import functools

import jax
import jax.numpy as jnp
from jax import lax
from jax.experimental import pallas as pl
from jax.experimental.pallas import tpu as pltpu
from jax.experimental.pallas import tpu_sc as plsc

F32 = jnp.float32
NCORES = 2
NSUB = 16


def _make_seg_sum(n_nodes, n_edges, dc, nch, edge_split, want_cnt, batch, nbuf,
                  idx_mul=1):
  stripe = (n_nodes // NSUB) // 8 * 8
  rem = n_nodes - stripe * NSUB
  assert 0 <= rem < stripe and rem % 8 == 0
  ept = (n_edges // NCORES if edge_split else n_edges) // NSUB
  nb = ept // batch
  assert nb % nbuf == 0 and batch % 8 == 0 and batch <= 128
  assert edge_split or batch % 16 == 0
  nchpc = 1 if edge_split else nch // NCORES
  nch_out = NCORES if edge_split else nch
  bpr = batch // 16

  mesh = plsc.VectorSubcoreMesh(
      core_axis_name="c", subcore_axis_name="s",
      num_cores=NCORES, num_subcores=NSUB)

  out_type = [jax.ShapeDtypeStruct((nch_out, n_nodes, dc), F32)]
  if want_cnt:
    out_type.append(jax.ShapeDtypeStruct((NSUB, n_nodes), F32))

  ngrp = nb // nbuf
  gb = nbuf * batch
  scratch = [
      pltpu.VMEM((ngrp, gb), jnp.int32),
      pltpu.VMEM((ngrp, gb), jnp.int32),
  ]
  if not edge_split:
    scratch.append(pltpu.VMEM((ngrp, gb), jnp.int32))
  scratch.append(pltpu.VMEM((2, gb, dc), F32))
  if want_cnt:
    scratch.append(pltpu.VMEM((n_nodes,), F32))
  scratch.extend([pltpu.SemaphoreType.DMA((2,))] * nbuf)
  scratch.append(pltpu.VMEM_SHARED((n_nodes, dc), F32))

  def body(*refs):
    it = iter(refs)
    tab = next(it); src2d = next(it); dst2d = next(it); zr = next(it)
    zc = next(it) if want_cnt else None
    out = next(it)
    cnt = next(it) if want_cnt else None
    src_v = next(it); dst_v = next(it)
    adj_v = None if edge_split else next(it)
    rows_v = next(it)
    cnt_v = next(it) if want_cnt else None
    sems = [next(it) for _ in range(nbuf)]
    acc = next(it)

    cid = lax.axis_index("c")
    sid = lax.axis_index("s")
    r0 = sid * stripe

    tix = sid + (cid * NSUB if edge_split else 0)
    pltpu.sync_copy(src2d.at[tix], src_v)
    pltpu.sync_copy(dst2d.at[tix], dst_v)
    if want_cnt:
      @pl.when(cid == 0)
      def _():
        pltpu.sync_copy(zc, cnt_v)
        ones16 = jnp.full((16,), 1.0, F32)

        def cnt_row(g, _):
          for v in range(gb // 16):
            plsc.addupdate_scatter(
                cnt_v, [dst_v[g, pl.ds(v * 16, 16)]], ones16)
          return 0

        lax.fori_loop(0, ngrp, cnt_row, 0)
        pltpu.sync_copy(cnt_v, cnt.at[sid])

    for j in range(nchpc):
      if edge_split:
        gref = src_v
      else:
        chunk = cid * nchpc + j
        base = ((chunk // idx_mul) * (idx_mul * n_nodes) + chunk % idx_mul)

        def adj_row(g, _):
          for v in range(gb // 16):
            adj_v[g, pl.ds(v * 16, 16)] = (
                src_v[g, pl.ds(v * 16, 16)] * idx_mul + base)
          return 0

        lax.fori_loop(0, ngrp, adj_row, 0)
        gref = adj_v

      pltpu.sync_copy(zr, acc.at[pl.ds(r0, stripe)])
      if rem:
        @pl.when(sid == 0)
        def _():
          pltpu.sync_copy(zr.at[pl.ds(0, rem)],
                          acc.at[pl.ds(NSUB * stripe, rem)])
      plsc.subcore_barrier()

      def issue(g, u, bk):
        pltpu.async_copy(tab.at[gref.at[g, pl.ds(u * batch, batch)]],
                         rows_v.at[bk, pl.ds(u * batch, batch)],
                         sems[u].at[bk])

      def wait_g(g, u, bk):
        pltpu.make_async_copy(tab.at[gref.at[g, pl.ds(u * batch, batch)]],
                              rows_v.at[bk, pl.ds(u * batch, batch)],
                              sems[u].at[bk]).wait()

      for u in range(nbuf):
        issue(0, u, 0)
        issue(1, u, 1)

      def group_body(g, _):
        bk = lax.rem(g, 2)
        for u in range(nbuf):
          wait_g(g, u, bk)
        pltpu.sync_copy(rows_v.at[bk], acc.at[dst_v.at[g]], add=True)

        @pl.when(g + 2 < ngrp)
        def _():
          for u in range(nbuf):
            issue(g + 2, u, bk)
        return 0

      lax.fori_loop(0, ngrp, group_body, 0)
      plsc.subcore_barrier()

      oc = cid if edge_split else chunk
      pltpu.sync_copy(acc.at[pl.ds(r0, stripe)],
                      out.at[oc, pl.ds(r0, stripe)])
      if rem:
        @pl.when(sid == 0)
        def _():
          pltpu.sync_copy(acc.at[pl.ds(NSUB * stripe, rem)],
                          out.at[oc, pl.ds(NSUB * stripe, rem)])

  return pl.kernel(body, out_type=tuple(out_type), mesh=mesh,
                   scratch_types=scratch,
                   compiler_params=pltpu.CompilerParams(
                       use_tc_tiling_on_sc=False,
                       needs_layout_passes=False))


def _layer0_tc(agg, cnt16, x, wl, bl, wr, mb):
  n, in_c = x.shape
  hid = wr.shape[1]
  nch_in, _, dca = agg.shape

  def kbody(agg_r, cnt_r, x_r, wl_r, bl_r, wr_r, h_r):
    inv = 1.0 / jnp.maximum(jnp.sum(cnt_r[0], 0), 1.0)[:, None]
    h = jnp.dot(x_r[...], wr_r[...], preferred_element_type=F32)
    for c in range(nch_in):
      h = h + jnp.dot(agg_r[c] * inv, wl_r[c], preferred_element_type=F32)
    h_r[...] = jnp.maximum(h + bl_r[...], 0.0)

  return pl.pallas_call(
      kbody,
      grid=(n // mb,),
      in_specs=[
          pl.BlockSpec((nch_in, mb, dca), lambda m: (0, m, 0)),
          pl.BlockSpec((1, NSUB, mb), lambda m: (m, 0, 0)),
          pl.BlockSpec((mb, in_c), lambda m: (m, 0)),
          pl.BlockSpec((nch_in, dca, hid), lambda m: (0, 0, 0)),
          pl.BlockSpec((1, hid), lambda m: (0, 0)),
          pl.BlockSpec((in_c, hid), lambda m: (0, 0)),
      ],
      out_specs=pl.BlockSpec((mb, hid), lambda m: (m, 0)),
      out_shape=jax.ShapeDtypeStruct((n, hid), F32),
  )(agg, cnt16, x, wl, bl, wr)


def _layer1_tc(agg, cnt16, h, wl, bl, wr, wp, mb):
  ncha, n, dca = agg.shape
  hid = wl.shape[2]
  pw = wp.shape[1]

  def kbody(agg_r, cnt_r, h_r, wl_r, bl_r, wr_r, wp_r, out2_r, h2_r, p_r):
    inv = 1.0 / jnp.maximum(jnp.sum(cnt_r[0], 0), 1.0)[:, None]
    o = jnp.dot(h_r[...], wr_r[...], preferred_element_type=F32)
    for c in range(ncha):
      o = o + jnp.dot(agg_r[c] * inv, wl_r[c], preferred_element_type=F32)
    o = o + bl_r[...]
    out2_r[...] = o
    h2 = jnp.maximum(o, 0.0)
    h2_r[...] = h2
    p_r[...] = jnp.dot(h2, wp_r[...], preferred_element_type=F32)

  return pl.pallas_call(
      kbody,
      grid=(n // mb,),
      in_specs=[
          pl.BlockSpec((ncha, mb, dca), lambda m: (0, m, 0)),
          pl.BlockSpec((1, NSUB, mb), lambda m: (m, 0, 0)),
          pl.BlockSpec((mb, hid), lambda m: (m, 0)),
          pl.BlockSpec((ncha, dca, hid), lambda m: (0, 0, 0)),
          pl.BlockSpec((1, hid), lambda m: (0, 0)),
          pl.BlockSpec((hid, hid), lambda m: (0, 0)),
          pl.BlockSpec((hid, pw), lambda m: (0, 0)),
      ],
      out_specs=[
          pl.BlockSpec((mb, hid), lambda m: (m, 0)),
          pl.BlockSpec((mb, hid), lambda m: (m, 0)),
          pl.BlockSpec((mb, pw), lambda m: (m, 0)),
      ],
      out_shape=[
          jax.ShapeDtypeStruct((n, hid), F32),
          jax.ShapeDtypeStruct((n, hid), F32),
          jax.ShapeDtypeStruct((n, pw), F32),
      ],
  )(agg, cnt16, h, wl, bl, wr, wp)


def _final_tc(parts, cnt16, h2, wr2p, bl2p, out_c, mb):
  n, hid = h2.shape
  pw = wr2p.shape[1]

  def kbody(parts_r, cnt_r, h2_r, wr_r, bl_r, logp_r, logits_r):
    inv = 1.0 / jnp.maximum(jnp.sum(cnt_r[0], 0), 1.0)[:, None]
    l = (parts_r[0] + parts_r[1]) * inv
    l = l + jnp.dot(h2_r[...], wr_r[...], preferred_element_type=F32)
    l = l + bl_r[...]
    l47 = l[:, :out_c]
    m = jnp.max(l47, axis=-1, keepdims=True)
    e = jnp.exp(l47 - m)
    sm = jnp.sum(e, axis=-1, keepdims=True)
    logp_r[...] = l47 - m - jnp.log(sm)
    logits_r[...] = l47

  return pl.pallas_call(
      kbody,
      grid=(n // mb,),
      in_specs=[
          pl.BlockSpec((2, mb, pw), lambda m: (0, m, 0)),
          pl.BlockSpec((1, NSUB, mb), lambda m: (m, 0, 0)),
          pl.BlockSpec((mb, hid), lambda m: (m, 0)),
          pl.BlockSpec((hid, pw), lambda m: (0, 0)),
          pl.BlockSpec((1, pw), lambda m: (0, 0)),
      ],
      out_specs=[
          pl.BlockSpec((mb, out_c), lambda m: (m, 0)),
          pl.BlockSpec((mb, out_c), lambda m: (m, 0)),
      ],
      out_shape=[
          jax.ShapeDtypeStruct((n, out_c), F32),
          jax.ShapeDtypeStruct((n, out_c), F32),
      ],
  )(parts, cnt16, h2, wr2p, bl2p)


@jax.jit
def kernel(x, edge_index, Wl0, bl0, Wr0, Wl1, bl1, Wr1, Wl2, bl2, Wr2):
  n, in_c = x.shape
  e = edge_index.shape[1]
  hid = Wl1.shape[0]
  out_c = Wl2.shape[1]
  pw = 48
  mb = 2000

  src = edge_index[0]
  dst = edge_index[1]

  b01, b2, nbf = 80, 40, 5
  src2d = src.reshape(NSUB, e // NSUB // (nbf * b01), nbf * b01)
  dst2d = dst.reshape(NSUB, e // NSUB // (nbf * b01), nbf * b01)
  nt2 = NCORES * NSUB
  src2d_2 = src.reshape(nt2, e // nt2 // (nbf * b2), nbf * b2)
  dst2d_2 = dst.reshape(nt2, e // nt2 // (nbf * b2), nbf * b2)

  stripe = (n // NSUB) // 8 * 8
  z64 = jnp.zeros((stripe, 64), F32)
  z32 = jnp.zeros((stripe, 32), F32)
  z48 = jnp.zeros((stripe, pw), F32)
  z1d = jnp.zeros((n,), F32)

  nch0 = 4 * (in_c // 128)
  seg0 = _make_seg_sum(n, e, 32, nch0, edge_split=False, want_cnt=True,
                       batch=b01, nbuf=5, idx_mul=nch0)
  agg0, cntp = seg0(x.reshape(nch0 * n, 32), src2d, dst2d, z32, z1d)
  cntp = cntp.reshape(NSUB, n // mb, mb).transpose(1, 0, 2)
  h = _layer0_tc(agg0, cntp, x, Wl0.reshape(nch0, 32, hid),
                 bl0.reshape(1, hid), Wr0, mb)

  nch1 = 2 * (hid // 128)
  seg1 = _make_seg_sum(n, e, 64, nch1, edge_split=False, want_cnt=False,
                       batch=b01, nbuf=5, idx_mul=nch1)
  (agg1,) = seg1(h.reshape(nch1 * n, 64), src2d, dst2d, z64)
  wl2p = jnp.pad(Wl2, ((0, 0), (0, pw - out_c)))
  out2, h2, p = _layer1_tc(agg1, cntp, h, Wl1.reshape(nch1, 64, hid),
                           bl1.reshape(1, hid), Wr1, wl2p, mb)

  seg2 = _make_seg_sum(n, e, pw, 1, edge_split=True, want_cnt=False,
                       batch=b2, nbuf=5)
  (agg2,) = seg2(p, src2d_2, dst2d_2, z48)
  wr2p = jnp.pad(Wr2, ((0, 0), (0, pw - out_c)))
  bl2p = jnp.pad(bl2, (0, pw - out_c)).reshape(1, pw)
  logp, logits = _final_tc(agg2, cntp, h2, wr2p, bl2p, out_c, mb)

  return logp, out2, h2, logits


# speedup vs baseline: 1.1451x; 1.1451x over previous
import functools

import jax
import jax.numpy as jnp
from jax import lax
from jax.experimental import pallas as pl
from jax.experimental.pallas import tpu as pltpu
from jax.experimental.pallas import tpu_sc as plsc

F32 = jnp.float32
NCORES = 2
NSUB = 16


def _make_seg_sum(n_nodes, n_edges, dc, nch, edge_split, want_cnt, batch, nbuf,
                  idx_mul=1):
  stripe = (n_nodes // NSUB) // 8 * 8
  rem = n_nodes - stripe * NSUB
  assert 0 <= rem < stripe and rem % 8 == 0
  ept = (n_edges // NCORES if edge_split else n_edges) // NSUB
  nb = ept // batch
  assert nb % nbuf == 0 and batch % 8 == 0 and batch <= 128
  assert edge_split or batch % 16 == 0
  nchpc = 1 if edge_split else nch // NCORES
  nch_out = NCORES if edge_split else nch
  bpr = batch // 16

  mesh = plsc.VectorSubcoreMesh(
      core_axis_name="c", subcore_axis_name="s",
      num_cores=NCORES, num_subcores=NSUB)

  out_type = [jax.ShapeDtypeStruct((nch_out, n_nodes, dc), F32)]
  if want_cnt:
    out_type.append(jax.ShapeDtypeStruct((n_nodes, 16), F32))

  scratch = [
      pltpu.VMEM((nb, batch), jnp.int32),
      pltpu.VMEM((nb, batch), jnp.int32),
  ]
  if not edge_split:
    scratch.append(pltpu.VMEM((nb, batch), jnp.int32))
  scratch.append(pltpu.VMEM((nbuf, batch, dc), F32))
  if want_cnt:
    scratch.append(pltpu.VMEM((batch, 16), F32))
  scratch.extend([pltpu.SemaphoreType.DMA] * nbuf)
  scratch.append(pltpu.VMEM_SHARED((n_nodes, dc), F32))
  if want_cnt:
    scratch.append(pltpu.VMEM_SHARED((n_nodes, 16), F32))

  def body(*refs):
    it = iter(refs)
    tab = next(it); src2d = next(it); dst2d = next(it); zr = next(it)
    if want_cnt:
      zc = next(it); ones1 = next(it)
    out = next(it)
    cnt = next(it) if want_cnt else None
    src_v = next(it); dst_v = next(it)
    adj_v = None if edge_split else next(it)
    rows_v = next(it)
    ones_v = next(it) if want_cnt else None
    sems = [next(it) for _ in range(nbuf)]
    acc = next(it)
    cntacc = next(it) if want_cnt else None

    cid = lax.axis_index("c")
    sid = lax.axis_index("s")
    r0 = sid * stripe

    tix = sid + (cid * NSUB if edge_split else 0)
    pltpu.sync_copy(src2d.at[tix], src_v)
    pltpu.sync_copy(dst2d.at[tix], dst_v)
    if want_cnt:
      pltpu.sync_copy(ones1, ones_v)

    for j in range(nchpc):
      do_cnt = want_cnt and j == 0
      if edge_split:
        gref = src_v
      else:
        chunk = cid * nchpc + j
        base = ((chunk // idx_mul) * (idx_mul * n_nodes) + chunk % idx_mul)

        def adj_row(i, _):
          for v in range(bpr):
            adj_v[i, pl.ds(v * 16, 16)] = (
                src_v[i, pl.ds(v * 16, 16)] * idx_mul + base)
          return 0

        lax.fori_loop(0, nb, adj_row, 0)
        gref = adj_v

      pltpu.sync_copy(zr, acc.at[pl.ds(r0, stripe)])
      if rem:
        @pl.when(sid == 0)
        def _():
          pltpu.sync_copy(zr.at[pl.ds(0, rem)],
                          acc.at[pl.ds(NSUB * stripe, rem)])
      if do_cnt:
        @pl.when(cid == 0)
        def _():
          pltpu.sync_copy(zc, cntacc.at[pl.ds(r0, stripe)])
        if rem:
          @pl.when((cid == 0) & (sid == 0))
          def _():
            pltpu.sync_copy(zc.at[pl.ds(0, rem)],
                            cntacc.at[pl.ds(NSUB * stripe, rem)])
      plsc.subcore_barrier()

      def issue(b, s):
        pltpu.async_copy(tab.at[gref.at[b]], rows_v.at[s], sems[s])

      for s in range(nbuf):
        issue(s, s)

      def round_body(i, _):
        for s in range(nbuf):
          b = i * nbuf + s
          pltpu.make_async_copy(tab.at[gref.at[b]], rows_v.at[s],
                                sems[s]).wait()
          pltpu.sync_copy(rows_v.at[s], acc.at[dst_v.at[b]], add=True)
          if do_cnt:
            @pl.when(cid == 0)
            def _():
              pltpu.sync_copy(ones_v, cntacc.at[dst_v.at[b]], add=True)

          @pl.when(b + nbuf < nb)
          def _():
            issue(b + nbuf, s)
        return 0

      lax.fori_loop(0, nb // nbuf, round_body, 0)
      plsc.subcore_barrier()

      oc = cid if edge_split else chunk
      pltpu.sync_copy(acc.at[pl.ds(r0, stripe)],
                      out.at[oc, pl.ds(r0, stripe)])
      if rem:
        @pl.when(sid == 0)
        def _():
          pltpu.sync_copy(acc.at[pl.ds(NSUB * stripe, rem)],
                          out.at[oc, pl.ds(NSUB * stripe, rem)])
      if do_cnt:
        @pl.when(cid == 0)
        def _():
          pltpu.sync_copy(cntacc.at[pl.ds(r0, stripe)],
                          cnt.at[pl.ds(r0, stripe)])
        if rem:
          @pl.when((cid == 0) & (sid == 0))
          def _():
            pltpu.sync_copy(cntacc.at[pl.ds(NSUB * stripe, rem)],
                            cnt.at[pl.ds(NSUB * stripe, rem)])

  return pl.kernel(body, out_type=tuple(out_type), mesh=mesh,
                   scratch_types=scratch,
                   compiler_params=pltpu.CompilerParams(
                       use_tc_tiling_on_sc=False))


def _layer0_tc(agg3, cnt16, x, wl, bl, wr, mb):
  n, in_c = x.shape
  hid = wr.shape[1]
  nch_in, _, dca = agg3.shape
  nch_out = hid // 128

  def kbody(agg_r, cnt_r, x_r, wl_r, bl_r, wr_r, h3_r):
    inv = 1.0 / jnp.maximum(cnt_r[:, 0:1], 1.0)
    h = jnp.dot(x_r[...], wr_r[...], preferred_element_type=F32)
    for c in range(nch_in):
      h = h + jnp.dot(agg_r[c] * inv, wl_r[c], preferred_element_type=F32)
    h = jnp.maximum(h + bl_r[...], 0.0)
    for c in range(nch_out):
      h3_r[c] = h[:, c * 128:(c + 1) * 128]

  return pl.pallas_call(
      kbody,
      grid=(n // mb,),
      in_specs=[
          pl.BlockSpec((nch_in, mb, dca), lambda m: (0, m, 0)),
          pl.BlockSpec((mb, 16), lambda m: (m, 0)),
          pl.BlockSpec((mb, in_c), lambda m: (m, 0)),
          pl.BlockSpec((nch_in, dca, hid), lambda m: (0, 0, 0)),
          pl.BlockSpec((1, hid), lambda m: (0, 0)),
          pl.BlockSpec((in_c, hid), lambda m: (0, 0)),
      ],
      out_specs=pl.BlockSpec((nch_out, mb, 128), lambda m: (0, m, 0)),
      out_shape=jax.ShapeDtypeStruct((nch_out, n, 128), F32),
  )(agg3, cnt16, x, wl, bl, wr)


def _layer1_tc(agg3, cnt16, h3, wl, bl, wr, wp, mb):
  nch, n, _ = h3.shape
  ncha, _, dca = agg3.shape
  hid = wl.shape[2]
  pw = wp.shape[1]

  def kbody(agg_r, cnt_r, h3_r, wl_r, bl_r, wr_r, wp_r, out2_r, h2_r, p_r):
    inv = 1.0 / jnp.maximum(cnt_r[:, 0:1], 1.0)
    o = jnp.dot(h3_r[0], wr_r[0], preferred_element_type=F32)
    for c in range(1, nch):
      o = o + jnp.dot(h3_r[c], wr_r[c], preferred_element_type=F32)
    for c in range(ncha):
      o = o + jnp.dot(agg_r[c] * inv, wl_r[c], preferred_element_type=F32)
    o = o + bl_r[...]
    out2_r[...] = o
    h2 = jnp.maximum(o, 0.0)
    h2_r[...] = h2
    p_r[...] = jnp.dot(h2, wp_r[...], preferred_element_type=F32)

  return pl.pallas_call(
      kbody,
      grid=(n // mb,),
      in_specs=[
          pl.BlockSpec((ncha, mb, dca), lambda m: (0, m, 0)),
          pl.BlockSpec((mb, 16), lambda m: (m, 0)),
          pl.BlockSpec((nch, mb, 128), lambda m: (0, m, 0)),
          pl.BlockSpec((ncha, dca, hid), lambda m: (0, 0, 0)),
          pl.BlockSpec((1, hid), lambda m: (0, 0)),
          pl.BlockSpec((nch, 128, hid), lambda m: (0, 0, 0)),
          pl.BlockSpec((hid, pw), lambda m: (0, 0)),
      ],
      out_specs=[
          pl.BlockSpec((mb, hid), lambda m: (m, 0)),
          pl.BlockSpec((mb, hid), lambda m: (m, 0)),
          pl.BlockSpec((mb, pw), lambda m: (m, 0)),
      ],
      out_shape=[
          jax.ShapeDtypeStruct((n, hid), F32),
          jax.ShapeDtypeStruct((n, hid), F32),
          jax.ShapeDtypeStruct((n, pw), F32),
      ],
  )(agg3, cnt16, h3, wl, bl, wr, wp)


def _final_tc(parts, cnt16, h2, wr2p, bl2p, out_c, mb):
  n, hid = h2.shape
  pw = wr2p.shape[1]

  def kbody(parts_r, cnt_r, h2_r, wr_r, bl_r, logp_r, logits_r):
    inv = 1.0 / jnp.maximum(cnt_r[:, 0:1], 1.0)
    l = (parts_r[0] + parts_r[1]) * inv
    l = l + jnp.dot(h2_r[...], wr_r[...], preferred_element_type=F32)
    l = l + bl_r[...]
    l47 = l[:, :out_c]
    m = jnp.max(l47, axis=-1, keepdims=True)
    e = jnp.exp(l47 - m)
    s = jnp.sum(e, axis=-1, keepdims=True)
    logp_r[...] = l47 - m - jnp.log(s)
    logits_r[...] = l47

  return pl.pallas_call(
      kbody,
      grid=(n // mb,),
      in_specs=[
          pl.BlockSpec((2, mb, pw), lambda m: (0, m, 0)),
          pl.BlockSpec((mb, 16), lambda m: (m, 0)),
          pl.BlockSpec((mb, hid), lambda m: (m, 0)),
          pl.BlockSpec((hid, pw), lambda m: (0, 0)),
          pl.BlockSpec((1, pw), lambda m: (0, 0)),
      ],
      out_specs=[
          pl.BlockSpec((mb, out_c), lambda m: (m, 0)),
          pl.BlockSpec((mb, out_c), lambda m: (m, 0)),
      ],
      out_shape=[
          jax.ShapeDtypeStruct((n, out_c), F32),
          jax.ShapeDtypeStruct((n, out_c), F32),
      ],
  )(parts, cnt16, h2, wr2p, bl2p)


@jax.jit
def kernel(x, edge_index, Wl0, bl0, Wr0, Wl1, bl1, Wr1, Wl2, bl2, Wr2):
  n, in_c = x.shape
  e = edge_index.shape[1]
  hid = Wl1.shape[0]
  out_c = Wl2.shape[1]
  pw = 48
  mb = 2000

  src = edge_index[0]
  dst = edge_index[1]

  b01, b2 = 80, 40
  src2d = src.reshape(NSUB, e // NSUB // b01, b01)
  dst2d = dst.reshape(NSUB, e // NSUB // b01, b01)
  nt2 = NCORES * NSUB
  src2d_2 = src.reshape(nt2, e // nt2 // b2, b2)
  dst2d_2 = dst.reshape(nt2, e // nt2 // b2, b2)

  stripe = (n // NSUB) // 8 * 8
  z64 = jnp.zeros((stripe, 64), F32)
  z48 = jnp.zeros((stripe, pw), F32)
  z16 = jnp.zeros((stripe, 16), F32)
  ones1 = jnp.ones((b01, 16), F32)

  nch0 = in_c // 128
  x3 = x.reshape(n, nch0, 128).transpose(1, 0, 2).reshape(2 * nch0 * n, 64)
  seg0 = _make_seg_sum(n, e, 64, 2 * nch0, edge_split=False, want_cnt=True,
                       batch=b01, nbuf=5, idx_mul=2)
  agg0, cnt16 = seg0(x3, src2d, dst2d, z64, z16, ones1)
  h3 = _layer0_tc(agg0, cnt16, x,
                  Wl0.reshape(2 * nch0, 64, hid), bl0.reshape(1, hid),
                  Wr0, mb)

  nch1 = hid // 128
  seg1 = _make_seg_sum(n, e, 64, 2 * nch1, edge_split=False, want_cnt=False,
                       batch=b01, nbuf=5, idx_mul=2)
  (agg1,) = seg1(h3.reshape(2 * nch1 * n, 64), src2d, dst2d, z64)
  wl2p = jnp.pad(Wl2, ((0, 0), (0, pw - out_c)))
  out2, h2, p = _layer1_tc(agg1, cnt16, h3,
                           Wl1.reshape(2 * nch1, 64, hid), bl1.reshape(1, hid),
                           Wr1.reshape(nch1, 128, hid), wl2p, mb)

  seg2 = _make_seg_sum(n, e, pw, 1, edge_split=True, want_cnt=False,
                       batch=b2, nbuf=5)
  (agg2,) = seg2(p, src2d_2, dst2d_2, z48)
  wr2p = jnp.pad(Wr2, ((0, 0), (0, pw - out_c)))
  bl2p = jnp.pad(bl2, (0, pw - out_c)).reshape(1, pw)
  logp, logits = _final_tc(agg2, cnt16, h2, wr2p, bl2p,
                           out_c, mb)

  return logp, out2, h2, logits
